# baseline (speedup 1.0000x reference)
import jax
import jax.numpy as jnp
from jax.experimental import pallas as pl

B = 2048
D = 2048
NP = 8
PE = 256
NC = 16
EPS = 1e-5

BB = 512

_DNT = (((1,), (1,)), ((), ()))


def _ln(v):
    n = v.shape[1]
    ones = jnp.ones((1, n), jnp.float32)
    ones_c = jnp.ones((n, 1), jnp.float32)
    m = jax.lax.dot_general(v, ones, _DNT) * (1.0 / n)
    s2 = jax.lax.dot_general(v * v, ones, _DNT) * (1.0 / n)
    inv = jax.lax.rsqrt(s2 - m * m + EPS)
    a = jax.lax.dot_general(inv, ones_c, _DNT)
    b = jax.lax.dot_general(-m * inv, ones_c, _DNT)
    return v * a + b


def _hc_kernel(x_ref, wp_ref, wc_ref, wpp_ref, wcp_ref, bp_ref, bpp_ref,
               bc_ref, bcp_ref, pl_out, cl_out, pp_out, cp_out):
    xb = x_ref[...]
    pl0 = jax.lax.dot_general(xb, wp_ref[...], _DNT) + bp_ref[...]
    cl_all = jax.lax.dot_general(xb, wc_ref[...], _DNT)

    pln = _ln(pl0)
    pl_out[...] = pln
    pp_out[...] = jax.lax.dot_general(pln, wpp_ref[...], _DNT) + bpp_ref[...]

    pc = jnp.argmax(pl0, axis=1)[:, None]
    lane8 = jax.lax.broadcasted_iota(jnp.int32, (1, NP), 1)
    onehot8 = (pc == lane8).astype(jnp.float32)
    lane128 = jax.lax.broadcasted_iota(jnp.int32, (1, NP * NC), 1)
    mask128 = pc == (lane128 // NC)

    frow = jax.lax.broadcasted_iota(jnp.int32, (NC, NP * NC), 0)
    fcol = jax.lax.broadcasted_iota(jnp.int32, (NC, NP * NC), 1)
    fold = (fcol % NC == frow).astype(jnp.float32)

    clm = jnp.where(mask128, cl_all, 0.0)
    cl16 = jax.lax.dot_general(clm, fold, _DNT)
    cl16 = cl16 + jax.lax.dot(onehot8, bc_ref[...])

    cln = _ln(cl16)
    cl_out[...] = cln

    rep = jax.lax.dot_general(cln, fold.T, _DNT)
    scat = jnp.where(mask128, rep, 0.0)
    cp = jax.lax.dot(scat, wcp_ref[...])
    cp_out[...] = cp + jax.lax.dot(onehot8, bcp_ref[...])


@jax.jit
def kernel(x, Wp, bp, Wpp, bpp, Wc, bc, Wcp, bcp):
    wc_flat = Wc.reshape(NP * NC, D)
    wcp_flat = jnp.transpose(Wcp, (0, 2, 1)).reshape(NP * NC, PE)

    grid = (B // BB,)
    out = pl.pallas_call(
        _hc_kernel,
        grid=grid,
        in_specs=[
            pl.BlockSpec((BB, D), lambda i: (i, 0)),
            pl.BlockSpec((NP, D), lambda i: (0, 0)),
            pl.BlockSpec((NP * NC, D), lambda i: (0, 0)),
            pl.BlockSpec((PE, NP), lambda i: (0, 0)),
            pl.BlockSpec((NP * NC, PE), lambda i: (0, 0)),
            pl.BlockSpec((1, NP), lambda i: (0, 0)),
            pl.BlockSpec((1, PE), lambda i: (0, 0)),
            pl.BlockSpec((NP, NC), lambda i: (0, 0)),
            pl.BlockSpec((NP, PE), lambda i: (0, 0)),
        ],
        out_specs=[
            pl.BlockSpec((BB, NP), lambda i: (i, 0)),
            pl.BlockSpec((BB, NC), lambda i: (i, 0)),
            pl.BlockSpec((BB, PE), lambda i: (i, 0)),
            pl.BlockSpec((BB, PE), lambda i: (i, 0)),
        ],
        out_shape=[
            jax.ShapeDtypeStruct((B, NP), jnp.float32),
            jax.ShapeDtypeStruct((B, NC), jnp.float32),
            jax.ShapeDtypeStruct((B, PE), jnp.float32),
            jax.ShapeDtypeStruct((B, PE), jnp.float32),
        ],
    )(x, Wp, wc_flat, Wpp, wcp_flat, bp[None, :], bpp[None, :], bc, bcp)
    return (out[0], out[1], out[2], out[3])
